# baseline (speedup 1.0000x reference)
import functools

import jax
import jax.numpy as jnp
from jax import lax
from jax.experimental import pallas as pl
from jax.experimental.pallas import tpu as pltpu
from jax.experimental.pallas import tpu_sc as plsc

N = 10000
E = 320000
SAT_DIM = 64
NEI_DIM = 64
HID = 128
OUT = 54

NC = 2
NS = 16
NW = NC * NS
L = 16

NPAD = 10240
SLICE = NPAD // NS
EW = E // NW
CW = 80
ROWS = EW // CW
EROWS = E // CW
_ST = 5
_SR = ROWS // _ST
NEG = -3.0e38

@functools.cache
def _mesh():
  return plsc.VectorSubcoreMesh(core_axis_name="c", subcore_axis_name="s",
                                num_cores=NC, num_subcores=NS)


def _worker_id():
  c = lax.axis_index("c")
  s = lax.axis_index("s")
  return c, s, s * NC + c


_RB = 400
_GRID = N // _RB


def _dot_t(a, w):
  return lax.dot_general(a, w, (((1,), (1,)), ((), ())),
                         preferred_element_type=jnp.float32)


def _k1_body(x_ref, wsat_ref, bsat_ref, wnei_ref, bnei_ref, w1_ref, a1_ref,
             hw_ref, asd_ref):
  xb = x_ref[...]
  sat = jnp.maximum(_dot_t(xb[:, :SAT_DIM], wsat_ref[...]) + bsat_ref[...], 0.0)
  nei = jnp.maximum(_dot_t(xb[:, SAT_DIM:], wnei_ref[...]) + bnei_ref[...], 0.0)
  h = sat + 0.5 * nei
  hw = _dot_t(h, w1_ref[...])
  hw_ref[...] = hw
  asd_ref[...] = jnp.dot(hw, a1_ref[...], preferred_element_type=jnp.float32)


def _full_spec(arr):
  nd = arr.ndim
  return pl.BlockSpec(arr.shape, lambda i, _n=nd: (0,) * _n)


def _tc_k1(x, wsat, bsat, wnei, bnei, w1, a1):
  return pl.pallas_call(
      _k1_body,
      grid=(_GRID,),
      in_specs=[pl.BlockSpec((_RB, SAT_DIM + NEI_DIM), lambda i: (i, 0))]
      + [_full_spec(a) for a in (wsat, bsat, wnei, bnei, w1, a1)],
      out_specs=[pl.BlockSpec((_RB, HID), lambda i: (i, 0)),
                 pl.BlockSpec((_RB, 2), lambda i: (i, 0))],
      out_shape=[jax.ShapeDtypeStruct((N, HID), jnp.float32),
                 jax.ShapeDtypeStruct((N, 2), jnp.float32)],
  )(x, wsat, bsat, wnei, bnei, w1, a1)


def _k2_body(u0_ref, u1_ref, d0_ref, d1_ref, es_ref, hw_ref, b_ref, w2_ref,
             a2_ref, hw2_ref, asd2_ref):
  es = es_ref[...]
  dtot = d0_ref[...] + d1_ref[...] + es
  o = (u0_ref[...] + u1_ref[...] + es * hw_ref[...]) / (dtot + 1e-16)
  h = jnp.maximum(o + b_ref[...], 0.0)
  hw2 = _dot_t(h, w2_ref[...])
  hw2_ref[...] = hw2
  asd2_ref[...] = jnp.dot(hw2, a2_ref[...], preferred_element_type=jnp.float32)


def _tc_k2(u0, u1, d0, d1, es, hw, b, w2, a2):
  return pl.pallas_call(
      _k2_body,
      grid=(_GRID,),
      in_specs=[pl.BlockSpec((_RB, HID), lambda i: (i, 0)),
                pl.BlockSpec((_RB, HID), lambda i: (i, 0)),
                pl.BlockSpec((_RB, 1), lambda i: (i, 0)),
                pl.BlockSpec((_RB, 1), lambda i: (i, 0)),
                pl.BlockSpec((_RB, 1), lambda i: (i, 0)),
                pl.BlockSpec((_RB, HID), lambda i: (i, 0)),
                _full_spec(b), _full_spec(w2), _full_spec(a2)],
      out_specs=[pl.BlockSpec((_RB, HID), lambda i: (i, 0)),
                 pl.BlockSpec((_RB, 2), lambda i: (i, 0))],
      out_shape=[jax.ShapeDtypeStruct((N, HID), jnp.float32),
                 jax.ShapeDtypeStruct((N, 2), jnp.float32)],
  )(u0, u1, d0, d1, es, hw, b, w2, a2)


def _k3_body(u0_ref, u1_ref, d0_ref, d1_ref, es_ref, hw_ref, b_ref,
             wf1_ref, bf1_ref, wf2_ref, bf2_ref, out_ref):
  es = es_ref[...]
  dtot = d0_ref[...] + d1_ref[...] + es
  o = (u0_ref[...] + u1_ref[...] + es * hw_ref[...]) / (dtot + 1e-16)
  h = jnp.maximum(o + b_ref[...], 0.0)
  hf = jnp.maximum(_dot_t(h, wf1_ref[...]) + bf1_ref[...], 0.0)
  out_ref[...] = _dot_t(hf, wf2_ref[...]) + bf2_ref[...]


def _tc_k3(u0, u1, d0, d1, es, hw, b, wf1, bf1, wf2, bf2):
  return pl.pallas_call(
      _k3_body,
      grid=(_GRID,),
      in_specs=[pl.BlockSpec((_RB, HID), lambda i: (i, 0)),
                pl.BlockSpec((_RB, HID), lambda i: (i, 0)),
                pl.BlockSpec((_RB, 1), lambda i: (i, 0)),
                pl.BlockSpec((_RB, 1), lambda i: (i, 0)),
                pl.BlockSpec((_RB, 1), lambda i: (i, 0)),
                pl.BlockSpec((_RB, HID), lambda i: (i, 0)),
                _full_spec(b), _full_spec(wf1), _full_spec(bf1),
                _full_spec(wf2), _full_spec(bf2)],
      out_specs=pl.BlockSpec((_RB, OUT), lambda i: (i, 0)),
      out_shape=jax.ShapeDtypeStruct((N, OUT), jnp.float32),
  )(u0, u1, d0, d1, es, hw, b, wf1, bf1, wf2, bf2)


def _lrelu(v):
  return jnp.where(v >= 0.0, v, 0.2 * v)


def _sc_a_body(asrc_h, adst_h, src_h, dst_h, e_h, m2_h,
               asrc_v, adst_v, src_v, dst_v, e_v, mloc, mred, red_v, msh):
  c, s, w = _worker_id()
  pltpu.sync_copy(asrc_h, asrc_v)
  pltpu.sync_copy(adst_h, adst_v)
  pltpu.sync_copy(src_h.at[w], src_v)
  pltpu.sync_copy(dst_h.at[w], dst_v)

  def init_body(i, _):
    mloc[pl.ds(i * L, L)] = jnp.full((L,), NEG, jnp.float32)
    return 0
  lax.fori_loop(0, NPAD // L, init_body, 0)

  def row_body(j, _):
    for k in range(CW // L):
      sv = src_v[j, pl.ds(k * L, L)]
      dv = dst_v[j, pl.ds(k * L, L)]
      av = plsc.load_gather(asrc_v, [sv])
      bv = plsc.load_gather(adst_v, [dv])
      e = _lrelu(av + bv)
      e_v[pl.ds(j * CW + k * L, L)] = e

      cur = plsc.load_gather(mloc, [dv])

      def wcond(mask):
        return jnp.any(mask)

      def wbody(mask):
        c0 = plsc.load_gather(mloc, [dv])
        plsc.store_scatter(mloc, [dv], jnp.maximum(c0, e), mask=mask)
        c1 = plsc.load_gather(mloc, [dv])
        return c1 < e

      lax.while_loop(wcond, wbody, cur < e)
    return 0
  lax.fori_loop(0, ROWS, row_body, 0)

  pltpu.sync_copy(e_v, e_h.at[pl.ds(w * EW, EW)])

  pltpu.sync_copy(mloc, msh.at[s, 0])
  plsc.subcore_barrier()
  pltpu.sync_copy(msh.at[:, 0, pl.ds(s * SLICE, SLICE)], red_v)

  def red_body(i, _):
    m = red_v[0, pl.ds(i * L, L)]
    for r in range(1, NS):
      m = jnp.maximum(m, red_v[r, pl.ds(i * L, L)])
    mred[pl.ds(i * L, L)] = m
    return 0
  lax.fori_loop(0, SLICE // L, red_body, 0)
  pltpu.sync_copy(mred, m2_h.at[c, 0, pl.ds(s * SLICE, SLICE)])


@functools.cache
def _sc_a():
  @functools.partial(
      pl.kernel,
      out_type=[jax.ShapeDtypeStruct((E,), jnp.float32),
                jax.ShapeDtypeStruct((NC, 1, NPAD), jnp.float32)],
      mesh=_mesh(),
      compiler_params=pltpu.CompilerParams(needs_layout_passes=False),
      scratch_types=[
          pltpu.VMEM((NPAD,), jnp.float32),
          pltpu.VMEM((NPAD,), jnp.float32),
          pltpu.VMEM((ROWS, CW), jnp.int32),
          pltpu.VMEM((ROWS, CW), jnp.int32),
          pltpu.VMEM((EW,), jnp.float32),
          pltpu.VMEM((NPAD,), jnp.float32),
          pltpu.VMEM((SLICE,), jnp.float32),
          pltpu.VMEM((NS, SLICE), jnp.float32),
          pltpu.VMEM_SHARED((NS, 1, NPAD), jnp.float32),
      ],
  )
  def sc_a(asrc_h, adst_h, src_h, dst_h, e_h, m2_h, *scratch):
    _sc_a_body(asrc_h, adst_h, src_h, dst_h, e_h, m2_h, *scratch)
  return sc_a


def _sc_b_body(e_h, dst_h, m2_h, asrc_h, adst_h, ee_h, d2_h, es_h,
               asrc_v, adst_v, mfull, t1, dst_v, ee_v, dloc,
               esl_v, dred, red_v, dsh):
  c, s, w = _worker_id()
  pltpu.sync_copy(asrc_h, asrc_v)
  pltpu.sync_copy(adst_h, adst_v)
  pltpu.sync_copy(m2_h.at[0, 0], mfull)
  pltpu.sync_copy(m2_h.at[1, 0], t1)
  pltpu.sync_copy(dst_h.at[w], dst_v)
  pltpu.sync_copy(e_h.at[pl.ds(w * EW, EW)], ee_v)

  def mf_body(i, _):
    sl = pl.ds(i * L, L)
    esf = _lrelu(asrc_v[sl] + adst_v[sl])
    mfull[sl] = jnp.maximum(jnp.maximum(mfull[sl], t1[sl]), esf)
    dloc[sl] = jnp.zeros((L,), jnp.float32)
    return 0
  lax.fori_loop(0, NPAD // L, mf_body, 0)

  def row_body(j, _):
    for k in range(CW // L):
      dv = dst_v[j, pl.ds(k * L, L)]
      e = ee_v[pl.ds(j * CW + k * L, L)]
      m = plsc.load_gather(mfull, [dv])
      ee = jnp.exp(e - m)
      ee_v[pl.ds(j * CW + k * L, L)] = ee
      plsc.addupdate_scatter(dloc, [dv], ee)
    return 0
  lax.fori_loop(0, ROWS, row_body, 0)

  pltpu.sync_copy(ee_v, ee_h.at[pl.ds(w * EW, EW)])

  def es_body(i, _):
    sl = pl.ds(s * SLICE + i * L, L)
    esf = _lrelu(asrc_v[sl] + adst_v[sl])
    esl_v[pl.ds(i * L, L)] = jnp.exp(esf - mfull[sl])
    return 0
  lax.fori_loop(0, SLICE // L, es_body, 0)

  @pl.when(c == 0)
  def _():
    pltpu.sync_copy(esl_v, es_h.at[pl.ds(s * SLICE, SLICE)])

  pltpu.sync_copy(dloc, dsh.at[s, 0])
  plsc.subcore_barrier()
  pltpu.sync_copy(dsh.at[:, 0, pl.ds(s * SLICE, SLICE)], red_v)

  def red_body(i, _):
    m = red_v[0, pl.ds(i * L, L)]
    for r in range(1, NS):
      m = m + red_v[r, pl.ds(i * L, L)]
    dred[pl.ds(i * L, L)] = m
    return 0
  lax.fori_loop(0, SLICE // L, red_body, 0)
  pltpu.sync_copy(dred, d2_h.at[c, 0, pl.ds(s * SLICE, SLICE)])


@functools.cache
def _sc_b():
  @functools.partial(
      pl.kernel,
      out_type=[jax.ShapeDtypeStruct((E,), jnp.float32),
                jax.ShapeDtypeStruct((NC, 1, NPAD), jnp.float32),
                jax.ShapeDtypeStruct((NPAD,), jnp.float32)],
      mesh=_mesh(),
      compiler_params=pltpu.CompilerParams(needs_layout_passes=False),
      scratch_types=[
          pltpu.VMEM((NPAD,), jnp.float32),
          pltpu.VMEM((NPAD,), jnp.float32),
          pltpu.VMEM((NPAD,), jnp.float32),
          pltpu.VMEM((NPAD,), jnp.float32),
          pltpu.VMEM((ROWS, CW), jnp.int32),
          pltpu.VMEM((EW,), jnp.float32),
          pltpu.VMEM((NPAD,), jnp.float32),
          pltpu.VMEM((SLICE,), jnp.float32),
          pltpu.VMEM((SLICE,), jnp.float32),
          pltpu.VMEM((NS, SLICE), jnp.float32),
          pltpu.VMEM_SHARED((NS, 1, NPAD), jnp.float32),
      ],
  )
  def sc_b(e_h, dst_h, m2_h, asrc_h, adst_h, ee_h, d2_h, es_h, *scratch):
    _sc_b_body(e_h, dst_h, m2_h, asrc_h, adst_h, ee_h, d2_h, es_h, *scratch)
  return sc_b


_ZR = 32


def _sc_c_body(ee_h, src_h, dst_h, hw_h, u_h,
               src_v, dst_v, ee_v, r0, r1, zbuf, g0, g1, s0, s1, ush):
  c, s, w = _worker_id()

  def z_body(i, _):
    for cc in range(HID // L):
      zbuf[i, pl.ds(cc * L, L)] = jnp.zeros((L,), jnp.float32)
    return 0
  lax.fori_loop(0, _ZR, z_body, 0)
  for t in range(SLICE // _ZR):
    pltpu.sync_copy(zbuf, ush.at[pl.ds(s * SLICE + t * _ZR, _ZR)])
  plsc.subcore_barrier()

  def scale(buf, j):
    def grp_body(g, _):
      ee16 = ee_v[pl.ds(j * CW + g * L, L)]

      def lane_body(i, _i):
        b = jnp.take_along_axis(ee16, jnp.full((L,), i, jnp.int32), axis=0)
        r = g * L + i
        for cc in range(HID // L):
          sl = pl.ds(cc * L, L)
          buf[r, sl] = buf[r, sl] * b
        return 0
      lax.fori_loop(0, L, lane_body, 0)
      return 0
    lax.fori_loop(0, CW // L, grp_body, 0)

  for st in range(_ST):
    pltpu.sync_copy(src_h.at[w * _ST + st], src_v.at[pl.ds(0, _SR)])
    pltpu.sync_copy(dst_h.at[w * _ST + st], dst_v)
    pltpu.sync_copy(ee_h.at[pl.ds(w * EW + st * _SR * CW, _SR * CW)], ee_v)
    for k in range(CW // L):
      src_v[_SR, pl.ds(k * L, L)] = jnp.zeros((L,), jnp.int32)

    pltpu.async_copy(hw_h.at[src_v.at[0]], r0, g0)
    pltpu.async_copy(hw_h.at[src_v.at[1]], r1, g1)

    def pair_body(t, _):
      j0 = 2 * t
      pltpu.make_async_copy(hw_h.at[src_v.at[j0]], r0, g0).wait()
      scale(r0, j0)
      sc0 = pltpu.async_copy(r0, ush.at[dst_v.at[j0]], s0, add=True)
      pltpu.make_async_copy(hw_h.at[src_v.at[j0 + 1]], r1, g1).wait()
      scale(r1, j0 + 1)
      sc1 = pltpu.async_copy(r1, ush.at[dst_v.at[j0 + 1]], s1, add=True)
      sc0.wait()
      pltpu.async_copy(hw_h.at[src_v.at[j0 + 2]], r0, g0)
      sc1.wait()
      pltpu.async_copy(hw_h.at[src_v.at[j0 + 3]], r1, g1)
      return 0
    lax.fori_loop(0, (_SR - 1) // 2, pair_body, 0)

    pltpu.make_async_copy(hw_h.at[src_v.at[_SR - 1]], r0, g0).wait()
    scale(r0, _SR - 1)
    pltpu.sync_copy(r0, ush.at[dst_v.at[_SR - 1]], add=True)
    pltpu.make_async_copy(hw_h.at[src_v.at[_SR]], r1, g1).wait()

  plsc.subcore_barrier()
  for t in range(SLICE // _ZR):
    sl = pl.ds(s * SLICE + t * _ZR, _ZR)
    pltpu.sync_copy(ush.at[sl], zbuf)
    pltpu.sync_copy(zbuf, u_h.at[c, sl])


@functools.cache
def _sc_c():
  @functools.partial(
      pl.kernel,
      out_type=jax.ShapeDtypeStruct((NC, NPAD, HID), jnp.float32),
      mesh=_mesh(),
      compiler_params=pltpu.CompilerParams(needs_layout_passes=False),
      scratch_types=[
          pltpu.VMEM((_SR + 1, CW), jnp.int32),
          pltpu.VMEM((_SR, CW), jnp.int32),
          pltpu.VMEM((_SR * CW,), jnp.float32),
          pltpu.VMEM((CW, HID), jnp.float32),
          pltpu.VMEM((CW, HID), jnp.float32),
          pltpu.VMEM((_ZR, HID), jnp.float32),
          pltpu.SemaphoreType.DMA,
          pltpu.SemaphoreType.DMA,
          pltpu.SemaphoreType.DMA,
          pltpu.SemaphoreType.DMA,
          pltpu.VMEM_SHARED((NPAD, HID), jnp.float32),
      ],
  )
  def sc_c(ee_h, src_h, dst_h, hw_h, u_h, *scratch):
    _sc_c_body(ee_h, src_h, dst_h, hw_h, u_h, *scratch)
  return sc_c


def _pad_n(v):
  return jnp.pad(v, (0, NPAD - N))


def kernel(x, edge_index, W_sat, b_sat, W_nei, b_nei, W1, a1_src, a1_dst, b1,
           W2, a2_src, a2_dst, b2, W_fc1, b_fc1, W_fc2, b_fc2):
  src = edge_index[0].astype(jnp.int32).reshape(NW, ROWS, CW)
  dst = edge_index[1].astype(jnp.int32).reshape(NW, ROWS, CW)
  src_c = src.reshape(NW * _ST, _SR, CW)
  dst_c = dst.reshape(NW * _ST, _SR, CW)

  a1 = jnp.stack([a1_src, a1_dst], axis=1)
  a2 = jnp.stack([a2_src, a2_dst], axis=1)

  hw1, asd1 = _tc_k1(x, W_sat, b_sat[None, :], W_nei, b_nei[None, :], W1, a1)

  def gat_edge_phase(hw, asd):
    asrc = _pad_n(asd[:, 0])
    adst = _pad_n(asd[:, 1])
    e, m2 = _sc_a()(asrc, adst, src, dst)
    ee, d2, es = _sc_b()(e, dst, m2, asrc, adst)
    u = _sc_c()(ee, src_c, dst_c, hw)
    return (u[0, :N], u[1, :N], d2[0, 0, :N, None], d2[1, 0, :N, None],
            es[:N, None])

  u0, u1, d0, d1, es = gat_edge_phase(hw1, asd1)
  hw2, asd2 = _tc_k2(u0, u1, d0, d1, es, hw1, b1[None, :], W2, a2)

  u0, u1, d0, d1, es = gat_edge_phase(hw2, asd2)
  return _tc_k3(u0, u1, d0, d1, es, hw2, b2[None, :], W_fc1, b_fc1[None, :],
                W_fc2, b_fc2[None, :])


# speedup vs baseline: 2.3132x; 2.3132x over previous
import functools

import jax
import jax.numpy as jnp
from jax import lax
from jax.experimental import pallas as pl
from jax.experimental.pallas import tpu as pltpu
from jax.experimental.pallas import tpu_sc as plsc

N = 10000
E = 320000
SAT_DIM = 64
NEI_DIM = 64
HID = 128
OUT = 54

NC = 2
NS = 16
NW = NC * NS
L = 16

NPAD = 10240
SLICE = NPAD // NS
EW = E // NW
CW = 80
ROWS = EW // CW
EROWS = E // CW
_ST = 5
_SR = ROWS // _ST
NEG = -3.0e38

@functools.cache
def _mesh():
  return plsc.VectorSubcoreMesh(core_axis_name="c", subcore_axis_name="s",
                                num_cores=NC, num_subcores=NS)


def _worker_id():
  c = lax.axis_index("c")
  s = lax.axis_index("s")
  return c, s, s * NC + c


_RB = 400
_GRID = N // _RB


def _dot_t(a, w):
  return lax.dot_general(a, w, (((1,), (1,)), ((), ())),
                         preferred_element_type=jnp.float32)


def _k1_body(x_ref, wsat_ref, bsat_ref, wnei_ref, bnei_ref, w1_ref, a1_ref,
             hw_ref, asd_ref):
  xb = x_ref[...]
  sat = jnp.maximum(_dot_t(xb[:, :SAT_DIM], wsat_ref[...]) + bsat_ref[...], 0.0)
  nei = jnp.maximum(_dot_t(xb[:, SAT_DIM:], wnei_ref[...]) + bnei_ref[...], 0.0)
  h = sat + 0.5 * nei
  hw = _dot_t(h, w1_ref[...])
  hw_ref[...] = hw
  asd_ref[...] = jnp.dot(hw, a1_ref[...], preferred_element_type=jnp.float32)


def _full_spec(arr):
  nd = arr.ndim
  return pl.BlockSpec(arr.shape, lambda i, _n=nd: (0,) * _n)


def _tc_k1(x, wsat, bsat, wnei, bnei, w1, a1):
  return pl.pallas_call(
      _k1_body,
      grid=(_GRID,),
      in_specs=[pl.BlockSpec((_RB, SAT_DIM + NEI_DIM), lambda i: (i, 0))]
      + [_full_spec(a) for a in (wsat, bsat, wnei, bnei, w1, a1)],
      out_specs=[pl.BlockSpec((_RB, HID), lambda i: (i, 0)),
                 pl.BlockSpec((_RB, 2), lambda i: (i, 0))],
      out_shape=[jax.ShapeDtypeStruct((N, HID), jnp.float32),
                 jax.ShapeDtypeStruct((N, 2), jnp.float32)],
  )(x, wsat, bsat, wnei, bnei, w1, a1)


def _k2_body(u0_ref, u1_ref, d0_ref, d1_ref, es_ref, hw_ref, b_ref, w2_ref,
             a2_ref, hw2_ref, asd2_ref):
  es = es_ref[...]
  dtot = d0_ref[...] + d1_ref[...] + es
  o = (u0_ref[...] + u1_ref[...] + es * hw_ref[...]) / (dtot + 1e-16)
  h = jnp.maximum(o + b_ref[...], 0.0)
  hw2 = _dot_t(h, w2_ref[...])
  hw2_ref[...] = hw2
  asd2_ref[...] = jnp.dot(hw2, a2_ref[...], preferred_element_type=jnp.float32)


def _tc_k2(u0, u1, d0, d1, es, hw, b, w2, a2):
  return pl.pallas_call(
      _k2_body,
      grid=(_GRID,),
      in_specs=[pl.BlockSpec((_RB, HID), lambda i: (i, 0)),
                pl.BlockSpec((_RB, HID), lambda i: (i, 0)),
                pl.BlockSpec((_RB, 1), lambda i: (i, 0)),
                pl.BlockSpec((_RB, 1), lambda i: (i, 0)),
                pl.BlockSpec((_RB, 1), lambda i: (i, 0)),
                pl.BlockSpec((_RB, HID), lambda i: (i, 0)),
                _full_spec(b), _full_spec(w2), _full_spec(a2)],
      out_specs=[pl.BlockSpec((_RB, HID), lambda i: (i, 0)),
                 pl.BlockSpec((_RB, 2), lambda i: (i, 0))],
      out_shape=[jax.ShapeDtypeStruct((N, HID), jnp.float32),
                 jax.ShapeDtypeStruct((N, 2), jnp.float32)],
  )(u0, u1, d0, d1, es, hw, b, w2, a2)


def _k3_body(u0_ref, u1_ref, d0_ref, d1_ref, es_ref, hw_ref, b_ref,
             wf1_ref, bf1_ref, wf2_ref, bf2_ref, out_ref):
  es = es_ref[...]
  dtot = d0_ref[...] + d1_ref[...] + es
  o = (u0_ref[...] + u1_ref[...] + es * hw_ref[...]) / (dtot + 1e-16)
  h = jnp.maximum(o + b_ref[...], 0.0)
  hf = jnp.maximum(_dot_t(h, wf1_ref[...]) + bf1_ref[...], 0.0)
  out_ref[...] = _dot_t(hf, wf2_ref[...]) + bf2_ref[...]


def _tc_k3(u0, u1, d0, d1, es, hw, b, wf1, bf1, wf2, bf2):
  return pl.pallas_call(
      _k3_body,
      grid=(_GRID,),
      in_specs=[pl.BlockSpec((_RB, HID), lambda i: (i, 0)),
                pl.BlockSpec((_RB, HID), lambda i: (i, 0)),
                pl.BlockSpec((_RB, 1), lambda i: (i, 0)),
                pl.BlockSpec((_RB, 1), lambda i: (i, 0)),
                pl.BlockSpec((_RB, 1), lambda i: (i, 0)),
                pl.BlockSpec((_RB, HID), lambda i: (i, 0)),
                _full_spec(b), _full_spec(wf1), _full_spec(bf1),
                _full_spec(wf2), _full_spec(bf2)],
      out_specs=pl.BlockSpec((_RB, OUT), lambda i: (i, 0)),
      out_shape=jax.ShapeDtypeStruct((N, OUT), jnp.float32),
  )(u0, u1, d0, d1, es, hw, b, wf1, bf1, wf2, bf2)


def _lrelu(v):
  return jnp.where(v >= 0.0, v, 0.2 * v)


def _sc_a_body(asrc_h, adst_h, src_h, dst_h, e_h, m2_h,
               asrc_v, adst_v, src_v, dst_v, e_v, mloc, mred, red_v, msh):
  c, s, w = _worker_id()
  pltpu.sync_copy(asrc_h, asrc_v)
  pltpu.sync_copy(adst_h, adst_v)
  pltpu.sync_copy(src_h.at[w], src_v)
  pltpu.sync_copy(dst_h.at[w], dst_v)

  def init_body(i, _):
    mloc[pl.ds(i * L, L)] = jnp.full((L,), NEG, jnp.float32)
    return 0
  lax.fori_loop(0, NPAD // L, init_body, 0)

  def row_body(j, _):
    for k in range(CW // L):
      sv = src_v[j, pl.ds(k * L, L)]
      dv = dst_v[j, pl.ds(k * L, L)]
      av = plsc.load_gather(asrc_v, [sv])
      bv = plsc.load_gather(adst_v, [dv])
      e = _lrelu(av + bv)
      e_v[pl.ds(j * CW + k * L, L)] = e

      cur = plsc.load_gather(mloc, [dv])

      def wcond(mask):
        return jnp.any(mask)

      def wbody(mask):
        c0 = plsc.load_gather(mloc, [dv])
        plsc.store_scatter(mloc, [dv], jnp.maximum(c0, e), mask=mask)
        c1 = plsc.load_gather(mloc, [dv])
        return c1 < e

      lax.while_loop(wcond, wbody, cur < e)
    return 0
  lax.fori_loop(0, ROWS, row_body, 0)

  pltpu.sync_copy(e_v, e_h.at[pl.ds(w * EW, EW)])

  pltpu.sync_copy(mloc, msh.at[s, 0])
  plsc.subcore_barrier()
  pltpu.sync_copy(msh.at[:, 0, pl.ds(s * SLICE, SLICE)], red_v)

  def red_body(i, _):
    m = red_v[0, pl.ds(i * L, L)]
    for r in range(1, NS):
      m = jnp.maximum(m, red_v[r, pl.ds(i * L, L)])
    mred[pl.ds(i * L, L)] = m
    return 0
  lax.fori_loop(0, SLICE // L, red_body, 0)
  pltpu.sync_copy(mred, m2_h.at[c, 0, pl.ds(s * SLICE, SLICE)])


@functools.cache
def _sc_a():
  @functools.partial(
      pl.kernel,
      out_type=[jax.ShapeDtypeStruct((E,), jnp.float32),
                jax.ShapeDtypeStruct((NC, 1, NPAD), jnp.float32)],
      mesh=_mesh(),
      compiler_params=pltpu.CompilerParams(needs_layout_passes=False),
      scratch_types=[
          pltpu.VMEM((NPAD,), jnp.float32),
          pltpu.VMEM((NPAD,), jnp.float32),
          pltpu.VMEM((ROWS, CW), jnp.int32),
          pltpu.VMEM((ROWS, CW), jnp.int32),
          pltpu.VMEM((EW,), jnp.float32),
          pltpu.VMEM((NPAD,), jnp.float32),
          pltpu.VMEM((SLICE,), jnp.float32),
          pltpu.VMEM((NS, SLICE), jnp.float32),
          pltpu.VMEM_SHARED((NS, 1, NPAD), jnp.float32),
      ],
  )
  def sc_a(asrc_h, adst_h, src_h, dst_h, e_h, m2_h, *scratch):
    _sc_a_body(asrc_h, adst_h, src_h, dst_h, e_h, m2_h, *scratch)
  return sc_a


def _sc_b_body(e_h, dst_h, m2_h, asrc_h, adst_h, ee_h, d2_h, es_h,
               asrc_v, adst_v, mfull, t1, dst_v, ee_v, dloc,
               esl_v, dred, red_v, dsh):
  c, s, w = _worker_id()
  pltpu.sync_copy(asrc_h, asrc_v)
  pltpu.sync_copy(adst_h, adst_v)
  pltpu.sync_copy(m2_h.at[0, 0], mfull)
  pltpu.sync_copy(m2_h.at[1, 0], t1)
  pltpu.sync_copy(dst_h.at[w], dst_v)
  pltpu.sync_copy(e_h.at[pl.ds(w * EW, EW)], ee_v)

  def mf_body(i, _):
    sl = pl.ds(i * L, L)
    esf = _lrelu(asrc_v[sl] + adst_v[sl])
    mfull[sl] = jnp.maximum(jnp.maximum(mfull[sl], t1[sl]), esf)
    dloc[sl] = jnp.zeros((L,), jnp.float32)
    return 0
  lax.fori_loop(0, NPAD // L, mf_body, 0)

  def row_body(j, _):
    for k in range(CW // L):
      dv = dst_v[j, pl.ds(k * L, L)]
      e = ee_v[pl.ds(j * CW + k * L, L)]
      m = plsc.load_gather(mfull, [dv])
      ee = jnp.exp(e - m)
      ee_v[pl.ds(j * CW + k * L, L)] = ee
      plsc.addupdate_scatter(dloc, [dv], ee)
    return 0
  lax.fori_loop(0, ROWS, row_body, 0)

  pltpu.sync_copy(ee_v, ee_h.at[pl.ds(w * EW, EW)])

  def es_body(i, _):
    sl = pl.ds(s * SLICE + i * L, L)
    esf = _lrelu(asrc_v[sl] + adst_v[sl])
    esl_v[pl.ds(i * L, L)] = jnp.exp(esf - mfull[sl])
    return 0
  lax.fori_loop(0, SLICE // L, es_body, 0)

  @pl.when(c == 0)
  def _():
    pltpu.sync_copy(esl_v, es_h.at[pl.ds(s * SLICE, SLICE)])

  pltpu.sync_copy(dloc, dsh.at[s, 0])
  plsc.subcore_barrier()
  pltpu.sync_copy(dsh.at[:, 0, pl.ds(s * SLICE, SLICE)], red_v)

  def red_body(i, _):
    m = red_v[0, pl.ds(i * L, L)]
    for r in range(1, NS):
      m = m + red_v[r, pl.ds(i * L, L)]
    dred[pl.ds(i * L, L)] = m
    return 0
  lax.fori_loop(0, SLICE // L, red_body, 0)
  pltpu.sync_copy(dred, d2_h.at[c, 0, pl.ds(s * SLICE, SLICE)])


@functools.cache
def _sc_b():
  @functools.partial(
      pl.kernel,
      out_type=[jax.ShapeDtypeStruct((E,), jnp.float32),
                jax.ShapeDtypeStruct((NC, 1, NPAD), jnp.float32),
                jax.ShapeDtypeStruct((NPAD,), jnp.float32)],
      mesh=_mesh(),
      compiler_params=pltpu.CompilerParams(needs_layout_passes=False),
      scratch_types=[
          pltpu.VMEM((NPAD,), jnp.float32),
          pltpu.VMEM((NPAD,), jnp.float32),
          pltpu.VMEM((NPAD,), jnp.float32),
          pltpu.VMEM((NPAD,), jnp.float32),
          pltpu.VMEM((ROWS, CW), jnp.int32),
          pltpu.VMEM((EW,), jnp.float32),
          pltpu.VMEM((NPAD,), jnp.float32),
          pltpu.VMEM((SLICE,), jnp.float32),
          pltpu.VMEM((SLICE,), jnp.float32),
          pltpu.VMEM((NS, SLICE), jnp.float32),
          pltpu.VMEM_SHARED((NS, 1, NPAD), jnp.float32),
      ],
  )
  def sc_b(e_h, dst_h, m2_h, asrc_h, adst_h, ee_h, d2_h, es_h, *scratch):
    _sc_b_body(e_h, dst_h, m2_h, asrc_h, adst_h, ee_h, d2_h, es_h, *scratch)
  return sc_b


_ZR = 32


def _sc_c_body(ee_h, src_h, dst_h, hw_h, u_h,
               src_v, dst_v, ee_v, r0, r1, zbuf, g0, g1, s0, s1, ush):
  c, s, w = _worker_id()

  def z_body(i, _):
    for cc in range(HID // L):
      zbuf[i, pl.ds(cc * L, L)] = jnp.zeros((L,), jnp.float32)
    return 0
  lax.fori_loop(0, _ZR, z_body, 0)
  for t in range(SLICE // _ZR):
    pltpu.sync_copy(zbuf, ush.at[pl.ds(s * SLICE + t * _ZR, _ZR)])
  plsc.subcore_barrier()

  def scale(buf, j):
    def grp_body(g, _):
      ee16 = ee_v[pl.ds(j * CW + g * L, L)]

      def lane_body(i, _i):
        b = jnp.take_along_axis(ee16, jnp.full((L,), i, jnp.int32), axis=0)
        r = g * L + i
        for cc in range(HID // L):
          sl = pl.ds(cc * L, L)
          buf[r, sl] = buf[r, sl] * b
        return 0
      lax.fori_loop(0, L, lane_body, 0)
      return 0
    lax.fori_loop(0, CW // L, grp_body, 0)

  for st in range(_ST):
    pltpu.sync_copy(src_h.at[w * _ST + st], src_v.at[pl.ds(0, _SR)])
    pltpu.sync_copy(dst_h.at[w * _ST + st], dst_v)
    pltpu.sync_copy(ee_h.at[pl.ds(w * EW + st * _SR * CW, _SR * CW)], ee_v)
    for k in range(CW // L):
      src_v[_SR, pl.ds(k * L, L)] = jnp.zeros((L,), jnp.int32)

    pltpu.async_copy(hw_h.at[src_v.at[0]], r0, g0)

    def pair_body(t, _):
      j0 = 2 * t
      pltpu.make_async_copy(hw_h.at[src_v.at[j0]], r0, g0).wait()
      pltpu.async_copy(hw_h.at[src_v.at[j0 + 1]], r1, g1)
      scale(r0, j0)
      pltpu.sync_copy(r0, ush.at[dst_v.at[j0]], add=True)
      pltpu.make_async_copy(hw_h.at[src_v.at[j0 + 1]], r1, g1).wait()
      pltpu.async_copy(hw_h.at[src_v.at[j0 + 2]], r0, g0)
      scale(r1, j0 + 1)
      pltpu.sync_copy(r1, ush.at[dst_v.at[j0 + 1]], add=True)
      return 0
    lax.fori_loop(0, (_SR - 1) // 2, pair_body, 0)

    pltpu.make_async_copy(hw_h.at[src_v.at[_SR - 1]], r0, g0).wait()
    scale(r0, _SR - 1)
    pltpu.sync_copy(r0, ush.at[dst_v.at[_SR - 1]], add=True)

  plsc.subcore_barrier()
  for t in range(SLICE // _ZR):
    sl = pl.ds(s * SLICE + t * _ZR, _ZR)
    pltpu.sync_copy(ush.at[sl], zbuf)
    pltpu.sync_copy(zbuf, u_h.at[c, sl])


@functools.cache
def _sc_c():
  @functools.partial(
      pl.kernel,
      out_type=jax.ShapeDtypeStruct((NC, NPAD, HID), jnp.float32),
      mesh=_mesh(),
      compiler_params=pltpu.CompilerParams(needs_layout_passes=False),
      scratch_types=[
          pltpu.VMEM((_SR + 1, CW), jnp.int32),
          pltpu.VMEM((_SR, CW), jnp.int32),
          pltpu.VMEM((_SR * CW,), jnp.float32),
          pltpu.VMEM((CW, HID), jnp.float32),
          pltpu.VMEM((CW, HID), jnp.float32),
          pltpu.VMEM((_ZR, HID), jnp.float32),
          pltpu.SemaphoreType.DMA,
          pltpu.SemaphoreType.DMA,
          pltpu.SemaphoreType.DMA,
          pltpu.SemaphoreType.DMA,
          pltpu.VMEM_SHARED((NPAD, HID), jnp.float32),
      ],
  )
  def sc_c(ee_h, src_h, dst_h, hw_h, u_h, *scratch):
    _sc_c_body(ee_h, src_h, dst_h, hw_h, u_h, *scratch)
  return sc_c


def _pad_n(v):
  return jnp.pad(v, (0, NPAD - N))


def kernel(x, edge_index, W_sat, b_sat, W_nei, b_nei, W1, a1_src, a1_dst, b1,
           W2, a2_src, a2_dst, b2, W_fc1, b_fc1, W_fc2, b_fc2):
  src = edge_index[0].astype(jnp.int32).reshape(NW, ROWS, CW)
  dst = edge_index[1].astype(jnp.int32).reshape(NW, ROWS, CW)
  src_c = src.reshape(NW * _ST, _SR, CW)
  dst_c = dst.reshape(NW * _ST, _SR, CW)

  a1 = jnp.stack([a1_src, a1_dst], axis=1)
  a2 = jnp.stack([a2_src, a2_dst], axis=1)

  hw1, asd1 = _tc_k1(x, W_sat, b_sat[None, :], W_nei, b_nei[None, :], W1, a1)

  def gat_edge_phase(hw, asd):
    asrc = _pad_n(asd[:, 0])
    adst = _pad_n(asd[:, 1])
    e, m2 = _sc_a()(asrc, adst, src, dst)
    ee, d2, es = _sc_b()(e, dst, m2, asrc, adst)
    u = _sc_c()(ee, src_c, dst_c, hw)
    return (u[0, :N], u[1, :N], d2[0, 0, :N, None], d2[1, 0, :N, None],
            es[:N, None])

  u0, u1, d0, d1, es = gat_edge_phase(hw1, asd1)
  hw2, asd2 = _tc_k2(u0, u1, d0, d1, es, hw1, b1[None, :], W2, a2)

  u0, u1, d0, d1, es = gat_edge_phase(hw2, asd2)
  return _tc_k3(u0, u1, d0, d1, es, hw2, b2[None, :], W_fc1, b_fc1[None, :],
                W_fc2, b_fc2[None, :])
